# Initial kernel scaffold; baseline (speedup 1.0000x reference)
import jax
import jax.numpy as jnp
from jax.experimental import pallas as pl


def kernel(node, X, edge_index, Wi, bi, W_in, b_in, W_h, b_h, W_out, b_out, W_lin, b_lin):
    raise NotImplementedError("write your pallas kernel here")


# speedup vs baseline: 3.9285x; 3.9285x over previous
import jax
import jax.numpy as jnp
from jax import lax
from jax.experimental import pallas as pl
from jax.experimental.pallas import tpu as pltpu
from jax.experimental.pallas import tpu_sc as plsc

N_NODES = 10000
D = 128
NC = 2
NS = 16
CHUNK = 128
N_PAD = 10240
ROWS_PER_SUB = N_PAD // NS
NINIT = ROWS_PER_SUB // CHUNK
NITER = 79
E_PAD = NC * NS * NITER * CHUNK
PER_CORE = NS * NITER


def _sc_pass(H, src_pad, dst_pad, zeros_blk, ones_blk, gather):
    out_type = jax.ShapeDtypeStruct((NC * N_PAD, D), jnp.float32)
    scratch = [
        pltpu.VMEM((CHUNK,), jnp.int32),
        pltpu.VMEM((CHUNK, D), jnp.float32),
        pltpu.VMEM_SHARED((N_PAD, D), jnp.float32),
        pltpu.SemaphoreType.DMA,
    ]
    if gather:
        scratch.append(pltpu.VMEM((CHUNK,), jnp.int32))

    mesh = plsc.VectorSubcoreMesh(core_axis_name="c", subcore_axis_name="s",
                                  num_cores=NC, num_subcores=NS)

    def body(H_hbm, src_hbm, dst_hbm, zeros_hbm, ones_hbm, out_hbm, *rest):
        if gather:
            dst_v, rows_v, acc_sh, sem, src_v = rest
        else:
            dst_v, rows_v, acc_sh, sem = rest
        c = lax.axis_index("c")
        s = lax.axis_index("s")
        r0 = s * ROWS_PER_SUB
        o0 = c * N_PAD + r0

        pltpu.sync_copy(zeros_hbm, rows_v)
        for k in range(NINIT):
            pltpu.sync_copy(rows_v, acc_sh.at[pl.ds(r0 + k * CHUNK, CHUNK)])
        if not gather:
            pltpu.sync_copy(ones_hbm, rows_v)
        plsc.subcore_barrier()

        def step(j, carry):
            chunk = c * PER_CORE + s + j * NS
            base = chunk * CHUNK
            pltpu.sync_copy(dst_hbm.at[pl.ds(base, CHUNK)], dst_v)
            if gather:
                pltpu.sync_copy(src_hbm.at[pl.ds(base, CHUNK)], src_v)
                pltpu.async_copy(H_hbm.at[src_v], rows_v, sem).wait()
            pltpu.sync_copy(rows_v, acc_sh.at[dst_v], add=True)
            return carry

        lax.fori_loop(0, NITER, step, 0, unroll=False)
        plsc.subcore_barrier()

        for k in range(NINIT):
            pltpu.sync_copy(acc_sh.at[pl.ds(r0 + k * CHUNK, CHUNK)], rows_v)
            pltpu.sync_copy(rows_v, out_hbm.at[pl.ds(o0 + k * CHUNK, CHUNK)])

    fn = pl.kernel(body, out_type=(out_type,), mesh=mesh,
                   scratch_types=scratch)
    (out,) = fn(H, src_pad, dst_pad, zeros_blk, ones_blk)
    return out.reshape(NC, N_PAD, D)


BLK = 2000


def _tc_layer(Gp, degp, W, b, act):
    def body(g_ref, d_ref, w_ref, b_ref, y_ref):
        g = g_ref[0] + g_ref[1]
        deg = jnp.maximum(d_ref[0, :, 0:1] + d_ref[1, :, 0:1], 1.0)
        y = jnp.dot(g / deg, w_ref[...], preferred_element_type=jnp.float32)
        y = y + b_ref[...]
        if act == "elu":
            y = jnp.where(y > 0, y, jnp.exp(y) - 1.0)
        else:
            y = jnp.maximum(y, 0.0)
        y_ref[...] = y

    return pl.pallas_call(
        body,
        grid=(N_NODES // BLK,),
        in_specs=[
            pl.BlockSpec((NC, BLK, D), lambda i: (0, i, 0)),
            pl.BlockSpec((NC, BLK, D), lambda i: (0, i, 0)),
            pl.BlockSpec((D, D), lambda i: (0, 0)),
            pl.BlockSpec((1, D), lambda i: (0, 0)),
        ],
        out_specs=pl.BlockSpec((BLK, D), lambda i: (i, 0)),
        out_shape=jax.ShapeDtypeStruct((N_NODES, D), jnp.float32),
    )(Gp, degp, W, b.reshape(1, D))


def _tc_layer3(Gp, degp, W, b):
    def body(g_ref, d_ref, w_ref, b_ref, y_ref, cs_ref):
        g = g_ref[0] + g_ref[1]
        deg = jnp.maximum(d_ref[0, :, 0:1] + d_ref[1, :, 0:1], 1.0)
        y = jnp.dot(g / deg, w_ref[...], preferred_element_type=jnp.float32)
        y = jnp.maximum(y + b_ref[...], 0.0)
        y_ref[...] = y
        part = jnp.sum(y.reshape(-1, 8, D), axis=0)

        @pl.when(pl.program_id(0) == 0)
        def _():
            cs_ref[...] = part

        @pl.when(pl.program_id(0) != 0)
        def _():
            cs_ref[...] += part

    return pl.pallas_call(
        body,
        grid=(N_NODES // BLK,),
        in_specs=[
            pl.BlockSpec((NC, BLK, D), lambda i: (0, i, 0)),
            pl.BlockSpec((NC, BLK, D), lambda i: (0, i, 0)),
            pl.BlockSpec((D, D), lambda i: (0, 0)),
            pl.BlockSpec((1, D), lambda i: (0, 0)),
        ],
        out_specs=[
            pl.BlockSpec((BLK, D), lambda i: (i, 0)),
            pl.BlockSpec((8, D), lambda i: (0, 0)),
        ],
        out_shape=[
            jax.ShapeDtypeStruct((N_NODES, D), jnp.float32),
            jax.ShapeDtypeStruct((8, D), jnp.float32),
        ],
    )(Gp, degp, W, b.reshape(1, D))


def _tc_decode(node, Wi, bi, W_lin, b_lin, colsum):
    B = node.shape[0]

    def body(n_ref, wi_ref, bi_ref, wl_ref, bl_ref, cs_ref, o_ref):
        x = jnp.dot(n_ref[...], wi_ref[...],
                    preferred_element_type=jnp.float32) + bi_ref[...]
        ymean = jnp.sum(cs_ref[...], axis=0, keepdims=True) * (1.0 / N_NODES)
        scal = jnp.dot(ymean, wl_ref[D:, :],
                       preferred_element_type=jnp.float32) + bl_ref[...]
        o_ref[...] = jnp.dot(x, wl_ref[0:D, :],
                             preferred_element_type=jnp.float32) + scal

    return pl.pallas_call(
        body,
        out_shape=jax.ShapeDtypeStruct((B, 1), jnp.float32),
    )(node, Wi, bi.reshape(1, D), W_lin, b_lin.reshape(1, 1), colsum)


def kernel(node, X, edge_index, Wi, bi, W_in, b_in, W_h, b_h, W_out, b_out,
           W_lin, b_lin):
    E = edge_index.shape[1]
    npad = E_PAD - E
    src_pad = jnp.concatenate(
        [edge_index[0], jnp.zeros((npad,), jnp.int32)])
    dst_pad = jnp.concatenate(
        [edge_index[1], jnp.full((npad,), N_NODES, jnp.int32)])
    zeros_blk = jnp.zeros((CHUNK, D), jnp.float32)
    ones_blk = jnp.ones((CHUNK, D), jnp.float32)

    degp = _sc_pass(X, src_pad, dst_pad, zeros_blk, ones_blk, gather=False)
    g1 = _sc_pass(X, src_pad, dst_pad, zeros_blk, ones_blk, gather=True)
    y1 = _tc_layer(g1, degp, W_in, b_in, act="elu")
    g2 = _sc_pass(y1, src_pad, dst_pad, zeros_blk, ones_blk, gather=True)
    y2 = _tc_layer(g2, degp, W_h, b_h, act="relu")
    g3 = _sc_pass(y2, src_pad, dst_pad, zeros_blk, ones_blk, gather=True)
    y3, colsum = _tc_layer3(g3, degp, W_out, b_out)
    del y3
    return _tc_decode(node, Wi, bi, W_lin, b_lin, colsum)
